# Initial kernel scaffold; baseline (speedup 1.0000x reference)
import jax
import jax.numpy as jnp
from jax.experimental import pallas as pl


def kernel(input_pos, kv_cache_k_0, kv_slice_k_0, kv_cache_v_0, kv_slice_v_0, kv_cache_k_1, kv_slice_k_1, kv_cache_v_1, kv_slice_v_1, kv_cache_k_2, kv_slice_k_2, kv_cache_v_2, kv_slice_v_2, kv_cache_k_3, kv_slice_k_3, kv_cache_v_3, kv_slice_v_3):
    raise NotImplementedError("write your pallas kernel here")


# speedup vs baseline: 1.1049x; 1.1049x over previous
import jax
import jax.numpy as jnp
from jax.experimental import pallas as pl
from jax.experimental.pallas import tpu as pltpu


def _scatter_body(pos_ref,
                  c0, c1, c2, c3, c4, c5, c6, c7,
                  ks0, vs0, ks1, vs1, ks2, vs2, ks3, vs3,
                  ok0, ov0, ok1, ov1, ok2, ov2, ok3, ov3,
                  vt0, vt1, vt2, vt3,
                  *sems):
    del c0, c1, c2, c3, c4, c5, c6, c7
    p = pos_ref[0]
    k_copies = []
    for i, (ks, ok) in enumerate(((ks0, ok0), (ks1, ok1), (ks2, ok2), (ks3, ok3))):
        c = pltpu.make_async_copy(ks, ok.at[:, :, pl.ds(p, 1), :], sems[i])
        c.start()
        k_copies.append(c)
    aligned = pl.multiple_of((p // 128) * 128, 128)
    col = p - aligned
    in_copies = []
    for i, (ov, vt) in enumerate(((ov0, vt0), (ov1, vt1), (ov2, vt2), (ov3, vt3))):
        c = pltpu.make_async_copy(ov.at[:, :, :, pl.ds(aligned, 128)], vt, sems[4 + i])
        c.start()
        in_copies.append(c)
    lane = jax.lax.broadcasted_iota(jnp.int32, vt0.shape, 3)
    out_copies = []
    for i, (vs, ov, vt) in enumerate(((vs0, ov0, vt0), (vs1, ov1, vt1),
                                      (vs2, ov2, vt2), (vs3, ov3, vt3))):
        in_copies[i].wait()
        vt[...] = jnp.where(lane == col, vs[...], vt[...])
        c = pltpu.make_async_copy(vt, ov.at[:, :, :, pl.ds(aligned, 128)], sems[4 + i])
        c.start()
        out_copies.append(c)
    for c in k_copies + out_copies:
        c.wait()


def kernel(input_pos, kv_cache_k_0, kv_slice_k_0, kv_cache_v_0, kv_slice_v_0, kv_cache_k_1, kv_slice_k_1, kv_cache_v_1, kv_slice_v_1, kv_cache_k_2, kv_slice_k_2, kv_cache_v_2, kv_slice_v_2, kv_cache_k_3, kv_slice_k_3, kv_cache_v_3, kv_slice_v_3):
    caches = (kv_cache_k_0, kv_cache_v_0, kv_cache_k_1, kv_cache_v_1,
              kv_cache_k_2, kv_cache_v_2, kv_cache_k_3, kv_cache_v_3)
    slices = (kv_slice_k_0, kv_slice_v_0, kv_slice_k_1, kv_slice_v_1,
              kv_slice_k_2, kv_slice_v_2, kv_slice_k_3, kv_slice_v_3)

    any_spec = pl.BlockSpec(memory_space=pltpu.HBM)
    vmem_spec = pl.BlockSpec(memory_space=pltpu.VMEM)
    smem_spec = pl.BlockSpec(memory_space=pltpu.SMEM)

    out = pl.pallas_call(
        _scatter_body,
        out_shape=tuple(jax.ShapeDtypeStruct(c.shape, c.dtype) for c in caches),
        in_specs=[smem_spec] + [any_spec] * 8 + [vmem_spec] * 8,
        out_specs=(any_spec,) * 8,
        scratch_shapes=[pltpu.VMEM((1, 4, 256, 128), jnp.float32)] * 4
                       + [pltpu.SemaphoreType.DMA] * 8,
        input_output_aliases={1 + i: i for i in range(8)},
        name="kv_cache_scatter_update",
    )(input_pos, *caches, *slices)

    ok0, ov0, ok1, ov1, ok2, ov2, ok3, ov3 = out
    return (ok0, ov0, ok1, ov1, ok2, ov2, ok3, ov3)
